# baseline (speedup 1.0000x reference)
import jax
import jax.numpy as jnp
from jax.experimental import pallas as pl
from jax.experimental.pallas import tpu as pltpu

_DIMS = (29, 256, 64, 32, 30)
_P = 4


def _block_diag(w, dtype):
    k, n = w.shape
    out = jnp.zeros((_P * k, _P * n), dtype)
    for j in range(_P):
        out = jax.lax.dynamic_update_slice(out, w.astype(dtype), (j * k, j * n))
    return out


def _mlp_kernel(x_ref, w1_ref, b1_ref, w2_ref, b2_ref, w3_ref, b3_ref,
                w4_ref, b4_ref, o_ref):
    h = x_ref[...].astype(jnp.bfloat16)

    def hidden(h, w_ref, b_ref):
        y = jnp.dot(h, w_ref[...], preferred_element_type=jnp.float32)
        return jnp.maximum(y.astype(jnp.bfloat16) + b_ref[...], 0)

    h = hidden(h, w1_ref, b1_ref)
    h = hidden(h, w2_ref, b2_ref)
    h = hidden(h, w3_ref, b3_ref)
    y = jnp.dot(h, w4_ref[...], preferred_element_type=jnp.float32)
    o_ref[...] = jnp.maximum(y + b4_ref[...], 0.0)


def kernel(x, w1, b1, w2, b2, w3, b3, w4, b4):
    batch, in_dim = x.shape
    assert in_dim == _DIMS[0]
    assert batch % (_P * 2048) == 0

    rows = batch // _P
    tbp = 2048
    grid = (rows // tbp,)

    xp = x.reshape(rows, _P * in_dim)

    ws = [_block_diag(w, jnp.bfloat16) for w in (w1, w2, w3, w4)]
    bs = [jnp.tile(b, _P).reshape(1, -1).astype(jnp.bfloat16)
          for b in (b1, b2, b3)]
    bs.append(jnp.tile(b4, _P).reshape(1, -1))

    x_spec = pl.BlockSpec((tbp, _P * in_dim), lambda i: (i, 0))
    out_spec = pl.BlockSpec((tbp, _P * _DIMS[-1]), lambda i: (i, 0))
    param_specs = []
    for w, b in zip(ws, bs):
        param_specs.append(pl.BlockSpec(w.shape, lambda i: (0, 0)))
        param_specs.append(pl.BlockSpec(b.shape, lambda i: (0, 0)))

    args = [xp]
    for w, b in zip(ws, bs):
        args.extend([w, b])

    flops = 2 * batch * sum(_DIMS[i] * _DIMS[i + 1] for i in range(4))
    bytes_accessed = 4 * batch * (_DIMS[0] + _DIMS[-1])

    outp = pl.pallas_call(
        _mlp_kernel,
        out_shape=jax.ShapeDtypeStruct((rows, _P * _DIMS[-1]), jnp.float32),
        grid=grid,
        in_specs=[x_spec] + param_specs,
        out_specs=out_spec,
        compiler_params=pltpu.CompilerParams(
            dimension_semantics=("parallel",)),
        cost_estimate=pl.CostEstimate(flops=flops, transcendentals=0,
                                      bytes_accessed=bytes_accessed),
    )(*args)
    return outp.reshape(batch, _DIMS[-1])


# speedup vs baseline: 2.6835x; 2.6835x over previous
import jax
import jax.numpy as jnp
from jax.experimental import pallas as pl
from jax.experimental.pallas import tpu as pltpu

_DIMS = (29, 256, 64, 32, 30)


def _mlp_kernel(x_ref, w1_ref, b1_ref, w2_ref, b2_ref, w3_ref, b3_ref,
                w4_ref, b4_ref, o_ref):
    h = x_ref[...].astype(jnp.bfloat16)

    def hidden(h, w_ref, b_ref):
        y = jnp.dot(w_ref[...], h, preferred_element_type=jnp.float32)
        return jnp.maximum(y.astype(jnp.bfloat16) + b_ref[...], 0)

    h = hidden(h, w1_ref, b1_ref)
    h = hidden(h, w2_ref, b2_ref)
    h = hidden(h, w3_ref, b3_ref)
    y = jnp.dot(w4_ref[...], h, preferred_element_type=jnp.float32)
    o_ref[...] = jnp.maximum(y + b4_ref[...], 0.0)


def kernel(x, w1, b1, w2, b2, w3, b3, w4, b4):
    batch, in_dim = x.shape
    assert in_dim == _DIMS[0]

    tbn = 2048
    assert batch % tbn == 0
    grid = (batch // tbn,)

    xt = x.T

    wts = [w.T.astype(jnp.bfloat16) for w in (w1, w2, w3, w4)]
    bts = [b.reshape(-1, 1).astype(jnp.bfloat16) for b in (b1, b2, b3)]
    bts.append(b4.reshape(-1, 1))

    x_spec = pl.BlockSpec((in_dim, tbn), lambda i: (0, i))
    out_spec = pl.BlockSpec((_DIMS[-1], tbn), lambda i: (0, i))
    param_specs = []
    for wt, bt in zip(wts, bts):
        param_specs.append(pl.BlockSpec(wt.shape, lambda i: (0, 0)))
        param_specs.append(pl.BlockSpec(bt.shape, lambda i: (0, 0)))

    args = [xt]
    for wt, bt in zip(wts, bts):
        args.extend([wt, bt])

    flops = 2 * batch * sum(_DIMS[i] * _DIMS[i + 1] for i in range(4))
    bytes_accessed = 4 * batch * (_DIMS[0] + _DIMS[-1])

    out_t = pl.pallas_call(
        _mlp_kernel,
        out_shape=jax.ShapeDtypeStruct((_DIMS[-1], batch), jnp.float32),
        grid=grid,
        in_specs=[x_spec] + param_specs,
        out_specs=out_spec,
        compiler_params=pltpu.CompilerParams(
            dimension_semantics=("parallel",)),
        cost_estimate=pl.CostEstimate(flops=flops, transcendentals=0,
                                      bytes_accessed=bytes_accessed),
    )(*args)
    return out_t.T


# speedup vs baseline: 4.2416x; 1.5806x over previous
import jax
import jax.numpy as jnp
from jax.experimental import pallas as pl
from jax.experimental.pallas import tpu as pltpu

_DIMS = (29, 256, 64, 32, 30)


def _mlp_kernel(x_ref, w1_ref, b1_ref, w2_ref, b2_ref, w3_ref, b3_ref,
                w4_ref, b4_ref, o_ref):
    h = x_ref[...].astype(jnp.bfloat16)

    def hidden(h, w_ref, b_ref):
        y = jnp.dot(w_ref[...], h, preferred_element_type=jnp.float32)
        return jnp.maximum(y.astype(jnp.bfloat16) + b_ref[...], 0)

    h = hidden(h, w1_ref, b1_ref)
    h = hidden(h, w2_ref, b2_ref)
    h = hidden(h, w3_ref, b3_ref)
    y = jnp.dot(w4_ref[...], h, preferred_element_type=jnp.float32)
    o_ref[...] = jnp.maximum(y + b4_ref[...], 0.0)


def kernel(x, w1, b1, w2, b2, w3, b3, w4, b4):
    batch, in_dim = x.shape
    assert in_dim == _DIMS[0]

    tbn = 8192
    assert batch % tbn == 0
    grid = (batch // tbn,)

    xt = x.T

    wts = [w.T.astype(jnp.bfloat16) for w in (w1, w2, w3, w4)]
    bts = [b.reshape(-1, 1).astype(jnp.bfloat16) for b in (b1, b2, b3)]
    bts.append(b4.reshape(-1, 1))

    x_spec = pl.BlockSpec((in_dim, tbn), lambda i: (0, i))
    out_spec = pl.BlockSpec((_DIMS[-1], tbn), lambda i: (0, i))
    param_specs = []
    for wt, bt in zip(wts, bts):
        param_specs.append(pl.BlockSpec(wt.shape, lambda i: (0, 0)))
        param_specs.append(pl.BlockSpec(bt.shape, lambda i: (0, 0)))

    args = [xt]
    for wt, bt in zip(wts, bts):
        args.extend([wt, bt])

    flops = 2 * batch * sum(_DIMS[i] * _DIMS[i + 1] for i in range(4))
    bytes_accessed = 4 * batch * (_DIMS[0] + _DIMS[-1])

    out_t = pl.pallas_call(
        _mlp_kernel,
        out_shape=jax.ShapeDtypeStruct((_DIMS[-1], batch), jnp.float32),
        grid=grid,
        in_specs=[x_spec] + param_specs,
        out_specs=out_spec,
        compiler_params=pltpu.CompilerParams(
            dimension_semantics=("parallel",)),
        cost_estimate=pl.CostEstimate(flops=flops, transcendentals=0,
                                      bytes_accessed=bytes_accessed),
    )(*args)
    return out_t.T


# speedup vs baseline: 4.3358x; 1.0222x over previous
import jax
import jax.numpy as jnp
from jax.experimental import pallas as pl
from jax.experimental.pallas import tpu as pltpu

_DIMS = (29, 256, 64, 32, 30)


def _mlp_kernel(x_ref, w1_ref, b1_ref, w2_ref, b2_ref, w3_ref, b3_ref,
                w4_ref, b4_ref, o_ref):
    h = x_ref[...].astype(jnp.bfloat16)

    def hidden(h, w_ref, b_ref):
        y = jnp.dot(w_ref[...], h, preferred_element_type=jnp.float32)
        return jnp.maximum(y.astype(jnp.bfloat16) + b_ref[...], 0)

    h = hidden(h, w1_ref, b1_ref)
    h = hidden(h, w2_ref, b2_ref)
    h = hidden(h, w3_ref, b3_ref)
    y = jnp.dot(w4_ref[...], h, preferred_element_type=jnp.float32)
    o_ref[...] = jnp.maximum(y + b4_ref[...], 0.0)


def kernel(x, w1, b1, w2, b2, w3, b3, w4, b4):
    batch, in_dim = x.shape
    assert in_dim == _DIMS[0]

    tbn = 16384
    assert batch % tbn == 0
    grid = (batch // tbn,)

    xt = x.T

    wts = [w.T.astype(jnp.bfloat16) for w in (w1, w2, w3, w4)]
    bts = [b.reshape(-1, 1).astype(jnp.bfloat16) for b in (b1, b2, b3)]
    bts.append(b4.reshape(-1, 1))

    x_spec = pl.BlockSpec((in_dim, tbn), lambda i: (0, i))
    out_spec = pl.BlockSpec((_DIMS[-1], tbn), lambda i: (0, i))
    param_specs = []
    for wt, bt in zip(wts, bts):
        param_specs.append(pl.BlockSpec(wt.shape, lambda i: (0, 0)))
        param_specs.append(pl.BlockSpec(bt.shape, lambda i: (0, 0)))

    args = [xt]
    for wt, bt in zip(wts, bts):
        args.extend([wt, bt])

    flops = 2 * batch * sum(_DIMS[i] * _DIMS[i + 1] for i in range(4))
    bytes_accessed = 4 * batch * (_DIMS[0] + _DIMS[-1])

    out_t = pl.pallas_call(
        _mlp_kernel,
        out_shape=jax.ShapeDtypeStruct((_DIMS[-1], batch), jnp.float32),
        grid=grid,
        in_specs=[x_spec] + param_specs,
        out_specs=out_spec,
        compiler_params=pltpu.CompilerParams(
            dimension_semantics=("parallel",)),
        cost_estimate=pl.CostEstimate(flops=flops, transcendentals=0,
                                      bytes_accessed=bytes_accessed),
    )(*args)
    return out_t.T


# speedup vs baseline: 4.3624x; 1.0061x over previous
import jax
import jax.numpy as jnp
from jax.experimental import pallas as pl
from jax.experimental.pallas import tpu as pltpu

_DIMS = (29, 256, 64, 32, 30)


def _mlp_kernel(x_ref, w1_ref, b1_ref, w2_ref, b2_ref, w3_ref, b3_ref,
                w4_ref, b4_ref, o_ref):
    h = x_ref[...].astype(jnp.bfloat16)

    def hidden(h, w_ref, b_ref):
        y = jnp.dot(w_ref[...], h, preferred_element_type=jnp.float32)
        return jnp.maximum(y.astype(jnp.bfloat16) + b_ref[...], 0)

    h = hidden(h, w1_ref, b1_ref)
    h = hidden(h, w2_ref, b2_ref)
    h = hidden(h, w3_ref, b3_ref)
    y = jnp.dot(w4_ref[...], h, preferred_element_type=jnp.float32)
    o_ref[...] = jnp.maximum(y + b4_ref[...], 0.0)


def kernel(x, w1, b1, w2, b2, w3, b3, w4, b4):
    batch, in_dim = x.shape
    assert in_dim == _DIMS[0]

    tbn = 32768
    assert batch % tbn == 0
    grid = (batch // tbn,)

    xt = x.T

    wts = [w.T.astype(jnp.bfloat16) for w in (w1, w2, w3, w4)]
    bts = [b.reshape(-1, 1).astype(jnp.bfloat16) for b in (b1, b2, b3)]
    bts.append(b4.reshape(-1, 1))

    x_spec = pl.BlockSpec((in_dim, tbn), lambda i: (0, i))
    out_spec = pl.BlockSpec((_DIMS[-1], tbn), lambda i: (0, i))
    param_specs = []
    for wt, bt in zip(wts, bts):
        param_specs.append(pl.BlockSpec(wt.shape, lambda i: (0, 0)))
        param_specs.append(pl.BlockSpec(bt.shape, lambda i: (0, 0)))

    args = [xt]
    for wt, bt in zip(wts, bts):
        args.extend([wt, bt])

    flops = 2 * batch * sum(_DIMS[i] * _DIMS[i + 1] for i in range(4))
    bytes_accessed = 4 * batch * (_DIMS[0] + _DIMS[-1])

    out_t = pl.pallas_call(
        _mlp_kernel,
        out_shape=jax.ShapeDtypeStruct((_DIMS[-1], batch), jnp.float32),
        grid=grid,
        in_specs=[x_spec] + param_specs,
        out_specs=out_spec,
        compiler_params=pltpu.CompilerParams(
            dimension_semantics=("parallel",)),
        cost_estimate=pl.CostEstimate(flops=flops, transcendentals=0,
                                      bytes_accessed=bytes_accessed),
    )(*args)
    return out_t.T
